# Initial kernel scaffold; baseline (speedup 1.0000x reference)
import jax
import jax.numpy as jnp
from jax.experimental import pallas as pl


def kernel(x):
    raise NotImplementedError("write your pallas kernel here")


# speedup vs baseline: 2.1749x; 2.1749x over previous
import math

import numpy as np
import jax
import jax.numpy as jnp
from jax.experimental import pallas as pl
from jax.experimental.pallas import tpu as pltpu


_Y_TABLE = np.array(
    [[16, 11, 10, 16, 24, 40, 51, 61],
     [12, 12, 14, 19, 26, 58, 60, 55],
     [14, 13, 16, 24, 40, 57, 69, 56],
     [14, 17, 22, 29, 51, 87, 80, 62],
     [18, 22, 37, 56, 68, 109, 103, 77],
     [24, 35, 55, 64, 81, 104, 113, 92],
     [49, 64, 78, 87, 103, 121, 120, 101],
     [72, 92, 95, 98, 112, 100, 103, 99]], dtype=np.float32).T

_C_TABLE = np.full((8, 8), 99.0, dtype=np.float32)
_C_TABLE[:4, :4] = np.array(
    [[17, 18, 24, 47],
     [18, 21, 26, 66],
     [24, 26, 56, 99],
     [47, 66, 99, 99]], dtype=np.float32).T

_C8 = np.array([[math.cos((2 * x + 1) * u * math.pi / 16) for x in range(8)]
                for u in range(8)], dtype=np.float32)
_A1 = np.array([1.0 / math.sqrt(2)] + [1.0] * 7, dtype=np.float32)
_SCALE8 = (np.outer(_A1, _A1) * 0.25).astype(np.float32)
_ALPHA8 = np.outer(_A1, _A1).astype(np.float32)


def _qfactor(quality):
    if quality < 50:
        q = 5000.0 / quality
    else:
        q = 200.0 - quality * 2
    return q / 100.0


def _blockdiag_c8(n):
    m = np.zeros((n, n), dtype=np.float32)
    for k in range(n // 8):
        m[8 * k:8 * k + 8, 8 * k:8 * k + 8] = _C8
    return m


def _pool_mats(th, tw):
    ah = np.zeros((th // 2, th), np.float32)
    ah[np.arange(th // 2), 2 * np.arange(th // 2)] = 0.5
    ah[np.arange(th // 2), 2 * np.arange(th // 2) + 1] = 0.5
    aw = np.zeros((tw, tw // 2), np.float32)
    aw[2 * np.arange(tw // 2), np.arange(tw // 2)] = 0.5
    aw[2 * np.arange(tw // 2) + 1, np.arange(tw // 2)] = 0.5
    return ah, aw


def _upsample_mats(th, tw):
    uh = np.zeros((th, th // 2), np.float32)
    uh[np.arange(th), np.arange(th) // 2] = 1.0
    uw = np.zeros((tw // 2, tw), np.float32)
    uw[np.arange(tw) // 2, np.arange(tw)] = 1.0
    return uh, uw


_CONST_CACHE = {}


def _consts(H, W, factor):
    key = (H, W, factor)
    if key in _CONST_CACHE:
        return _CONST_CACHE[key]
    f = np.float32(factor)
    h2, w2 = H // 2, W // 2
    bdy_l = _blockdiag_c8(H)
    bdy_r = _blockdiag_c8(W)
    bdc_l = _blockdiag_c8(h2)
    bdc_r = _blockdiag_c8(w2)
    scy = np.tile(_SCALE8, (H // 8, W // 8)).astype(np.float32)
    aly = np.tile(_ALPHA8, (H // 8, W // 8)).astype(np.float32)
    qy = np.tile(_Y_TABLE * f, (H // 8, W // 8)).astype(np.float32)
    scc = np.tile(_SCALE8, (h2 // 8, w2 // 8)).astype(np.float32)
    alc = np.tile(_ALPHA8, (h2 // 8, w2 // 8)).astype(np.float32)
    qc = np.tile(_C_TABLE * f, (h2 // 8, w2 // 8)).astype(np.float32)
    ah, aw = _pool_mats(H, W)
    uh, uw = _upsample_mats(H, W)
    out = tuple(jnp.asarray(a) for a in (
        bdy_l, np.ascontiguousarray(bdy_r.T), np.ascontiguousarray(bdy_l.T),
        bdy_r, scy, aly, qy,
        bdc_l, np.ascontiguousarray(bdc_r.T), np.ascontiguousarray(bdc_l.T),
        bdc_r, scc, alc, qc,
        ah, aw, uh, uw))
    _CONST_CACHE[key] = out
    return out


def _dot(a, b):
    return jnp.dot(a, b, preferred_element_type=jnp.float32)


def _dj_kernel(x_ref,
               yl_ref, yr_ref, yli_ref, yri_ref, scy_ref, aly_ref, qy_ref,
               cl_ref, cr_ref, cli_ref, cri_ref, scc_ref, alc_ref, qc_ref,
               ah_ref, aw_ref, uh_ref, uw_ref, o_ref):
    r = x_ref[0, 0]
    g = x_ref[0, 1]
    b = x_ref[0, 2]
    y = 0.299 * r + 0.587 * g + 0.114 * b
    cb = -0.168736 * r - 0.331264 * g + 0.5 * b + 128.0
    cr = 0.5 * r - 0.418688 * g - 0.081312 * b + 128.0

    ah = ah_ref[...]
    aw = aw_ref[...]
    pcb = _dot(_dot(ah, cb), aw)
    pcr = _dot(_dot(ah, cr), aw)

    d = _dot(_dot(yl_ref[...], y - 128.0), yr_ref[...]) * scy_ref[...]
    qy = qy_ref[...]
    coef = jnp.round(d / qy)
    deq = (coef * qy) * aly_ref[...]
    rec = _dot(_dot(yli_ref[...], deq), yri_ref[...])
    y_rec = 0.25 * rec + 128.0

    cl = cl_ref[...]
    crm = cr_ref[...]
    cli = cli_ref[...]
    cri = cri_ref[...]
    scc = scc_ref[...]
    alc = alc_ref[...]
    qc = qc_ref[...]

    def core(p):
        d = _dot(_dot(cl, p - 128.0), crm) * scc
        k = jnp.round(d / qc)
        dq = (k * qc) * alc
        return 0.25 * _dot(_dot(cli, dq), cri) + 128.0

    ocb = core(pcb)
    ocr = core(pcr)

    uh = uh_ref[...]
    uw = uw_ref[...]
    cbu = _dot(_dot(uh, ocb), uw) - 128.0
    cru = _dot(_dot(uh, ocr), uw) - 128.0
    o_ref[0, 0] = jnp.clip(y_rec + 1.402 * cru, 0.0, 255.0)
    o_ref[0, 1] = jnp.clip(y_rec - 0.344136 * cbu - 0.714136 * cru,
                           0.0, 255.0)
    o_ref[0, 2] = jnp.clip(y_rec + 1.772 * cbu, 0.0, 255.0)


def kernel(x):
    x = x.astype(jnp.float32)
    B, C, H, W = x.shape
    assert C == 3 and H % 16 == 0 and W % 16 == 0
    consts = _consts(H, W, float(_qfactor(80)))
    img_spec = pl.BlockSpec((1, 3, H, W), lambda b: (b, 0, 0, 0))
    const_specs = [pl.BlockSpec(c.shape, lambda b: (0, 0)) for c in consts]
    return pl.pallas_call(
        _dj_kernel,
        out_shape=jax.ShapeDtypeStruct((B, 3, H, W), jnp.float32),
        grid=(B,),
        in_specs=[img_spec] + const_specs,
        out_specs=img_spec,
        compiler_params=pltpu.CompilerParams(
            dimension_semantics=("parallel",),
            vmem_limit_bytes=48 * 1024 * 1024),
    )(x, *consts)


# speedup vs baseline: 2.3313x; 1.0719x over previous
import math

import numpy as np
import jax
import jax.numpy as jnp
from jax.experimental import pallas as pl
from jax.experimental.pallas import tpu as pltpu


_Y_TABLE = np.array(
    [[16, 11, 10, 16, 24, 40, 51, 61],
     [12, 12, 14, 19, 26, 58, 60, 55],
     [14, 13, 16, 24, 40, 57, 69, 56],
     [14, 17, 22, 29, 51, 87, 80, 62],
     [18, 22, 37, 56, 68, 109, 103, 77],
     [24, 35, 55, 64, 81, 104, 113, 92],
     [49, 64, 78, 87, 103, 121, 120, 101],
     [72, 92, 95, 98, 112, 100, 103, 99]], dtype=np.float32).T

_C_TABLE = np.full((8, 8), 99.0, dtype=np.float32)
_C_TABLE[:4, :4] = np.array(
    [[17, 18, 24, 47],
     [18, 21, 26, 66],
     [24, 26, 56, 99],
     [47, 66, 99, 99]], dtype=np.float32).T

_C8 = np.array([[math.cos((2 * x + 1) * u * math.pi / 16) for x in range(8)]
                for u in range(8)], dtype=np.float32)
_A1 = np.array([1.0 / math.sqrt(2)] + [1.0] * 7, dtype=np.float32)
_SCALE8 = (np.outer(_A1, _A1) * 0.25).astype(np.float32)
_ALPHA8 = np.outer(_A1, _A1).astype(np.float32)


def _qfactor(quality):
    if quality < 50:
        q = 5000.0 / quality
    else:
        q = 200.0 - quality * 2
    return q / 100.0


def _blockdiag_c8(n):
    m = np.zeros((n, n), dtype=np.float32)
    for k in range(n // 8):
        m[8 * k:8 * k + 8, 8 * k:8 * k + 8] = _C8
    return m


def _pool_mats(th, tw):
    ah = np.zeros((th // 2, th), np.float32)
    ah[np.arange(th // 2), 2 * np.arange(th // 2)] = 0.5
    ah[np.arange(th // 2), 2 * np.arange(th // 2) + 1] = 0.5
    aw = np.zeros((tw, tw // 2), np.float32)
    aw[2 * np.arange(tw // 2), np.arange(tw // 2)] = 0.5
    aw[2 * np.arange(tw // 2) + 1, np.arange(tw // 2)] = 0.5
    return ah, aw


def _upsample_mats(th, tw):
    uh = np.zeros((th, th // 2), np.float32)
    uh[np.arange(th), np.arange(th) // 2] = 1.0
    uw = np.zeros((tw // 2, tw), np.float32)
    uw[np.arange(tw) // 2, np.arange(tw)] = 1.0
    return uh, uw


_CONST_CACHE = {}


def _consts(H, W, factor):
    key = (H, W, factor)
    if key in _CONST_CACHE:
        return _CONST_CACHE[key]
    f = np.float32(factor)
    h2, w2 = H // 2, W // 2
    bdy_l = _blockdiag_c8(H)
    bdy_r = _blockdiag_c8(W)
    bdc_l = _blockdiag_c8(h2)
    bdc_r = _blockdiag_c8(w2)
    scy = np.tile(_SCALE8, (H // 8, W // 8)).astype(np.float32)
    aly = np.tile(_ALPHA8, (H // 8, W // 8)).astype(np.float32)
    qy = np.tile(_Y_TABLE * f, (H // 8, W // 8)).astype(np.float32)
    scc = np.tile(_SCALE8, (h2 // 8, w2 // 8)).astype(np.float32)
    alc = np.tile(_ALPHA8, (h2 // 8, w2 // 8)).astype(np.float32)
    qc = np.tile(_C_TABLE * f, (h2 // 8, w2 // 8)).astype(np.float32)
    ah, aw = _pool_mats(H, W)
    uh, uw = _upsample_mats(H, W)
    out = tuple(jnp.asarray(a) for a in (
        bdy_l, np.ascontiguousarray(bdy_r.T), np.ascontiguousarray(bdy_l.T),
        bdy_r, scy, aly, qy,
        bdc_l, np.ascontiguousarray(bdc_r.T), np.ascontiguousarray(bdc_l.T),
        bdc_r, scc, alc, qc,
        ah, aw, uh, uw))
    _CONST_CACHE[key] = out
    return out


def _dot(a, b):
    return jnp.dot(a, b, preferred_element_type=jnp.float32)


def _dj_kernel(x_ref,
               yl_ref, yr_ref, yli_ref, yri_ref, scy_ref, aly_ref, qy_ref,
               cl_ref, cr_ref, cli_ref, cri_ref, scc_ref, alc_ref, qc_ref,
               ah_ref, aw_ref, uh_ref, uw_ref, o_ref):
    ah = ah_ref[...]
    aw = aw_ref[...]
    qy = qy_ref[...]
    cl = cl_ref[...]
    crm = cr_ref[...]
    cli = cli_ref[...]
    cri = cri_ref[...]
    scc = scc_ref[...]
    alc = alc_ref[...]
    qc = qc_ref[...]
    uh = uh_ref[...]
    uw = uw_ref[...]

    def core(p):
        d = _dot(_dot(cl, p - 128.0), crm) * scc
        k = jnp.round(d / qc)
        dq = (k * qc) * alc
        return 0.25 * _dot(_dot(cli, dq), cri) + 128.0

    for i in range(x_ref.shape[0]):
        r = x_ref[i, 0]
        g = x_ref[i, 1]
        b = x_ref[i, 2]
        y = 0.299 * r + 0.587 * g + 0.114 * b
        cb = -0.168736 * r - 0.331264 * g + 0.5 * b + 128.0
        cr = 0.5 * r - 0.418688 * g - 0.081312 * b + 128.0

        pcb = _dot(_dot(ah, cb), aw)
        pcr = _dot(_dot(ah, cr), aw)

        d = _dot(_dot(yl_ref[...], y - 128.0), yr_ref[...]) * scy_ref[...]
        coef = jnp.round(d / qy)
        deq = (coef * qy) * aly_ref[...]
        rec = _dot(_dot(yli_ref[...], deq), yri_ref[...])
        y_rec = 0.25 * rec + 128.0

        ocb = core(pcb)
        ocr = core(pcr)

        cbu = _dot(_dot(uh, ocb), uw) - 128.0
        cru = _dot(_dot(uh, ocr), uw) - 128.0
        o_ref[i, 0] = jnp.clip(y_rec + 1.402 * cru, 0.0, 255.0)
        o_ref[i, 1] = jnp.clip(y_rec - 0.344136 * cbu - 0.714136 * cru,
                               0.0, 255.0)
        o_ref[i, 2] = jnp.clip(y_rec + 1.772 * cbu, 0.0, 255.0)


def kernel(x):
    x = x.astype(jnp.float32)
    B, C, H, W = x.shape
    assert C == 3 and H % 16 == 0 and W % 16 == 0
    consts = _consts(H, W, float(_qfactor(80)))
    bi = 2 if B % 2 == 0 else 1
    img_spec = pl.BlockSpec((bi, 3, H, W), lambda b: (b, 0, 0, 0))
    const_specs = [pl.BlockSpec(s.shape, lambda b: (0, 0)) for s in consts]
    return pl.pallas_call(
        _dj_kernel,
        out_shape=jax.ShapeDtypeStruct((B, 3, H, W), jnp.float32),
        grid=(B // bi,),
        in_specs=[img_spec] + const_specs,
        out_specs=img_spec,
        compiler_params=pltpu.CompilerParams(
            dimension_semantics=("parallel",),
            vmem_limit_bytes=48 * 1024 * 1024),
    )(x, *consts)


# speedup vs baseline: 2.4032x; 1.0308x over previous
import math

import numpy as np
import jax
import jax.numpy as jnp
from jax.experimental import pallas as pl
from jax.experimental.pallas import tpu as pltpu


_Y_TABLE = np.array(
    [[16, 11, 10, 16, 24, 40, 51, 61],
     [12, 12, 14, 19, 26, 58, 60, 55],
     [14, 13, 16, 24, 40, 57, 69, 56],
     [14, 17, 22, 29, 51, 87, 80, 62],
     [18, 22, 37, 56, 68, 109, 103, 77],
     [24, 35, 55, 64, 81, 104, 113, 92],
     [49, 64, 78, 87, 103, 121, 120, 101],
     [72, 92, 95, 98, 112, 100, 103, 99]], dtype=np.float32).T

_C_TABLE = np.full((8, 8), 99.0, dtype=np.float32)
_C_TABLE[:4, :4] = np.array(
    [[17, 18, 24, 47],
     [18, 21, 26, 66],
     [24, 26, 56, 99],
     [47, 66, 99, 99]], dtype=np.float32).T

_C8 = np.array([[math.cos((2 * x + 1) * u * math.pi / 16) for x in range(8)]
                for u in range(8)], dtype=np.float32)
_A1 = np.array([1.0 / math.sqrt(2)] + [1.0] * 7, dtype=np.float32)
_SCALE8 = (np.outer(_A1, _A1) * 0.25).astype(np.float32)
_ALPHA8 = np.outer(_A1, _A1).astype(np.float32)


def _qfactor(quality):
    if quality < 50:
        q = 5000.0 / quality
    else:
        q = 200.0 - quality * 2
    return q / 100.0


def _blockdiag_c8(n):
    m = np.zeros((n, n), dtype=np.float32)
    for k in range(n // 8):
        m[8 * k:8 * k + 8, 8 * k:8 * k + 8] = _C8
    return m


def _pool_mats(th, tw):
    ah = np.zeros((th // 2, th), np.float32)
    ah[np.arange(th // 2), 2 * np.arange(th // 2)] = 0.5
    ah[np.arange(th // 2), 2 * np.arange(th // 2) + 1] = 0.5
    aw = np.zeros((tw, tw // 2), np.float32)
    aw[2 * np.arange(tw // 2), np.arange(tw // 2)] = 0.5
    aw[2 * np.arange(tw // 2) + 1, np.arange(tw // 2)] = 0.5
    return ah, aw


def _upsample_mats(th, tw):
    uh = np.zeros((th, th // 2), np.float32)
    uh[np.arange(th), np.arange(th) // 2] = 1.0
    uw = np.zeros((tw // 2, tw), np.float32)
    uw[np.arange(tw) // 2, np.arange(tw)] = 1.0
    return uh, uw


_CONST_CACHE = {}


def _consts(H, W, factor):
    key = (H, W, factor)
    if key in _CONST_CACHE:
        return _CONST_CACHE[key]
    f = np.float32(factor)
    h2, w2 = H // 2, W // 2
    bdy_l = _blockdiag_c8(H)
    bdy_r = _blockdiag_c8(W)
    bdc_l = _blockdiag_c8(h2)
    bdc_r = _blockdiag_c8(w2)
    scy = np.tile(_SCALE8, (H // 8, W // 8)).astype(np.float32)
    aly = np.tile(_ALPHA8, (H // 8, W // 8)).astype(np.float32)
    qy = np.tile(_Y_TABLE * f, (H // 8, W // 8)).astype(np.float32)
    scc = np.tile(_SCALE8, (h2 // 8, w2 // 8)).astype(np.float32)
    alc = np.tile(_ALPHA8, (h2 // 8, w2 // 8)).astype(np.float32)
    qc = np.tile(_C_TABLE * f, (h2 // 8, w2 // 8)).astype(np.float32)
    ah, aw = _pool_mats(H, W)
    uh, uw = _upsample_mats(H, W)
    out = tuple(jnp.asarray(a) for a in (
        bdy_l, np.ascontiguousarray(bdy_r.T), np.ascontiguousarray(bdy_l.T),
        bdy_r, scy, aly, qy,
        bdc_l, np.ascontiguousarray(bdc_r.T), np.ascontiguousarray(bdc_l.T),
        bdc_r, scc, alc, qc,
        ah, aw, uh, uw))
    _CONST_CACHE[key] = out
    return out


def _dot(a, b):
    return jnp.dot(a, b, preferred_element_type=jnp.float32)


def _dj_kernel(x_ref,
               yl_ref, yr_ref, yli_ref, yri_ref, scy_ref, aly_ref, qy_ref,
               cl_ref, cr_ref, cli_ref, cri_ref, scc_ref, alc_ref, qc_ref,
               ah_ref, aw_ref, uh_ref, uw_ref, o_ref):
    ah = ah_ref[...]
    aw = aw_ref[...]
    qy = qy_ref[...]
    cl = cl_ref[...]
    crm = cr_ref[...]
    cli = cli_ref[...]
    cri = cri_ref[...]
    scc = scc_ref[...]
    alc = alc_ref[...]
    qc = qc_ref[...]
    uh = uh_ref[...]
    uw = uw_ref[...]

    def core(p):
        d = _dot(_dot(cl, p - 128.0), crm) * scc
        k = jnp.round(d / qc)
        dq = (k * qc) * alc
        return 0.25 * _dot(_dot(cli, dq), cri) + 128.0

    for i in range(x_ref.shape[0]):
        r = x_ref[i, 0]
        g = x_ref[i, 1]
        b = x_ref[i, 2]
        y = 0.299 * r + 0.587 * g + 0.114 * b
        cb = -0.168736 * r - 0.331264 * g + 0.5 * b + 128.0
        cr = 0.5 * r - 0.418688 * g - 0.081312 * b + 128.0

        pcb = _dot(_dot(ah, cb), aw)
        pcr = _dot(_dot(ah, cr), aw)

        d = _dot(_dot(yl_ref[...], y - 128.0), yr_ref[...]) * scy_ref[...]
        coef = jnp.round(d / qy)
        deq = (coef * qy) * aly_ref[...]
        rec = _dot(_dot(yli_ref[...], deq), yri_ref[...])
        y_rec = 0.25 * rec + 128.0

        ocb = core(pcb)
        ocr = core(pcr)

        cbu = _dot(_dot(uh, ocb), uw) - 128.0
        cru = _dot(_dot(uh, ocr), uw) - 128.0
        o_ref[i, 0] = jnp.clip(y_rec + 1.402 * cru, 0.0, 255.0)
        o_ref[i, 1] = jnp.clip(y_rec - 0.344136 * cbu - 0.714136 * cru,
                               0.0, 255.0)
        o_ref[i, 2] = jnp.clip(y_rec + 1.772 * cbu, 0.0, 255.0)


def kernel(x):
    x = x.astype(jnp.float32)
    B, C, H, W = x.shape
    assert C == 3 and H % 16 == 0 and W % 16 == 0
    consts = _consts(H, W, float(_qfactor(80)))
    bi = 4 if B % 4 == 0 else 1
    img_spec = pl.BlockSpec((bi, 3, H, W), lambda b: (b, 0, 0, 0))
    const_specs = [pl.BlockSpec(s.shape, lambda b: (0, 0)) for s in consts]
    return pl.pallas_call(
        _dj_kernel,
        out_shape=jax.ShapeDtypeStruct((B, 3, H, W), jnp.float32),
        grid=(B // bi,),
        in_specs=[img_spec] + const_specs,
        out_specs=img_spec,
        compiler_params=pltpu.CompilerParams(
            dimension_semantics=("parallel",),
            vmem_limit_bytes=48 * 1024 * 1024),
    )(x, *consts)


# speedup vs baseline: 2.6122x; 1.0870x over previous
import math

import numpy as np
import jax
import jax.numpy as jnp
from jax.experimental import pallas as pl
from jax.experimental.pallas import tpu as pltpu


_Y_TABLE = np.array(
    [[16, 11, 10, 16, 24, 40, 51, 61],
     [12, 12, 14, 19, 26, 58, 60, 55],
     [14, 13, 16, 24, 40, 57, 69, 56],
     [14, 17, 22, 29, 51, 87, 80, 62],
     [18, 22, 37, 56, 68, 109, 103, 77],
     [24, 35, 55, 64, 81, 104, 113, 92],
     [49, 64, 78, 87, 103, 121, 120, 101],
     [72, 92, 95, 98, 112, 100, 103, 99]], dtype=np.float32).T

_C_TABLE = np.full((8, 8), 99.0, dtype=np.float32)
_C_TABLE[:4, :4] = np.array(
    [[17, 18, 24, 47],
     [18, 21, 26, 66],
     [24, 26, 56, 99],
     [47, 66, 99, 99]], dtype=np.float32).T

_C8 = np.array([[math.cos((2 * x + 1) * u * math.pi / 16) for x in range(8)]
                for u in range(8)], dtype=np.float32)
_A1 = np.array([1.0 / math.sqrt(2)] + [1.0] * 7, dtype=np.float32)
_SCALE8 = (np.outer(_A1, _A1) * 0.25).astype(np.float32)
_ALPHA8 = np.outer(_A1, _A1).astype(np.float32)


def _qfactor(quality):
    if quality < 50:
        q = 5000.0 / quality
    else:
        q = 200.0 - quality * 2
    return q / 100.0


def _blockdiag_c8(n):
    m = np.zeros((n, n), dtype=np.float32)
    for k in range(n // 8):
        m[8 * k:8 * k + 8, 8 * k:8 * k + 8] = _C8
    return m


def _pool_mats(th, tw):
    ah = np.zeros((th // 2, th), np.float32)
    ah[np.arange(th // 2), 2 * np.arange(th // 2)] = 0.5
    ah[np.arange(th // 2), 2 * np.arange(th // 2) + 1] = 0.5
    aw = np.zeros((tw, tw // 2), np.float32)
    aw[2 * np.arange(tw // 2), np.arange(tw // 2)] = 0.5
    aw[2 * np.arange(tw // 2) + 1, np.arange(tw // 2)] = 0.5
    return ah, aw


def _upsample_mats(th, tw):
    uh = np.zeros((th, th // 2), np.float32)
    uh[np.arange(th), np.arange(th) // 2] = 1.0
    uw = np.zeros((tw // 2, tw), np.float32)
    uw[np.arange(tw) // 2, np.arange(tw)] = 1.0
    return uh, uw


_CONST_CACHE = {}


def _consts(H, W, factor):
    key = (H, W, factor)
    if key in _CONST_CACHE:
        return _CONST_CACHE[key]
    f = np.float32(factor)
    h2, w2 = H // 2, W // 2
    bdy_l = _blockdiag_c8(H)
    bdy_r = _blockdiag_c8(W)
    bdc_l = _blockdiag_c8(h2)
    bdc_r = _blockdiag_c8(w2)
    scy = np.tile(_SCALE8, (H // 8, W // 8)).astype(np.float32)
    aly = np.tile(_ALPHA8, (H // 8, W // 8)).astype(np.float32)
    qy = np.tile(_Y_TABLE * f, (H // 8, W // 8)).astype(np.float32)
    bds_l = _blockdiag_c8(2 * h2)
    scc = np.tile(_SCALE8, (2 * h2 // 8, w2 // 8)).astype(np.float32)
    alc = np.tile(_ALPHA8, (2 * h2 // 8, w2 // 8)).astype(np.float32)
    qc = np.tile(_C_TABLE * f, (2 * h2 // 8, w2 // 8)).astype(np.float32)
    ah, aw = _pool_mats(H, W)
    uh, uw = _upsample_mats(H, W)
    uh2 = np.zeros((2 * H, 2 * h2), np.float32)
    uh2[:H, :h2] = uh
    uh2[H:, h2:] = uh
    out = tuple(jnp.asarray(a) for a in (
        bdy_l, np.ascontiguousarray(bdy_r.T), np.ascontiguousarray(bdy_l.T),
        bdy_r, scy, aly, qy,
        bds_l, np.ascontiguousarray(bdc_r.T), np.ascontiguousarray(bds_l.T),
        bdc_r, scc, alc, qc,
        ah, aw, uh2, uw))
    _CONST_CACHE[key] = out
    return out


def _dot(a, b):
    return jnp.dot(a, b, preferred_element_type=jnp.float32)


def _dj_kernel(x_ref,
               yl_ref, yr_ref, yli_ref, yri_ref, scy_ref, aly_ref, qy_ref,
               cl_ref, cr_ref, cli_ref, cri_ref, scc_ref, alc_ref, qc_ref,
               ah_ref, aw_ref, uh_ref, uw_ref, o_ref):
    ah = ah_ref[...]
    aw = aw_ref[...]
    qy = qy_ref[...]
    cl = cl_ref[...]
    crm = cr_ref[...]
    cli = cli_ref[...]
    cri = cri_ref[...]
    scc = scc_ref[...]
    alc = alc_ref[...]
    qc = qc_ref[...]
    uh = uh_ref[...]
    uw = uw_ref[...]

    H = x_ref.shape[2]

    for i in range(x_ref.shape[0]):
        r = x_ref[i, 0]
        g = x_ref[i, 1]
        b = x_ref[i, 2]
        y = 0.299 * r + 0.587 * g + 0.114 * b
        cb = -0.168736 * r - 0.331264 * g + 0.5 * b + 128.0
        cr = 0.5 * r - 0.418688 * g - 0.081312 * b + 128.0

        pcb = _dot(_dot(ah, cb), aw)
        pcr = _dot(_dot(ah, cr), aw)

        d = _dot(_dot(yl_ref[...], y - 128.0), yr_ref[...]) * scy_ref[...]
        coef = jnp.round(d / qy)
        deq = (coef * qy) * aly_ref[...]
        rec = _dot(_dot(yli_ref[...], deq), yri_ref[...])
        y_rec = 0.25 * rec + 128.0

        p = jnp.concatenate([pcb, pcr], axis=0)
        dc = _dot(_dot(cl, p - 128.0), crm) * scc
        k = jnp.round(dc / qc)
        dq = (k * qc) * alc
        oc = 0.25 * _dot(_dot(cli, dq), cri) + 128.0

        u = _dot(_dot(uh, oc), uw) - 128.0
        cbu = u[:H]
        cru = u[H:]
        o_ref[i, 0] = jnp.clip(y_rec + 1.402 * cru, 0.0, 255.0)
        o_ref[i, 1] = jnp.clip(y_rec - 0.344136 * cbu - 0.714136 * cru,
                               0.0, 255.0)
        o_ref[i, 2] = jnp.clip(y_rec + 1.772 * cbu, 0.0, 255.0)


def kernel(x):
    x = x.astype(jnp.float32)
    B, C, H, W = x.shape
    assert C == 3 and H % 16 == 0 and W % 16 == 0
    consts = _consts(H, W, float(_qfactor(80)))
    bi = 4 if B % 4 == 0 else 1
    img_spec = pl.BlockSpec((bi, 3, H, W), lambda b: (b, 0, 0, 0))
    const_specs = [pl.BlockSpec(s.shape, lambda b: (0, 0)) for s in consts]
    return pl.pallas_call(
        _dj_kernel,
        out_shape=jax.ShapeDtypeStruct((B, 3, H, W), jnp.float32),
        grid=(B // bi,),
        in_specs=[img_spec] + const_specs,
        out_specs=img_spec,
        compiler_params=pltpu.CompilerParams(
            dimension_semantics=("parallel",),
            vmem_limit_bytes=48 * 1024 * 1024),
    )(x, *consts)
